# baseline (speedup 1.0000x reference)
import functools

import jax
import jax.numpy as jnp
from jax import lax
from jax.experimental import pallas as pl
from jax.experimental.pallas import tpu as pltpu
from jax.experimental.pallas import tpu_sc as plsc

_N = 10000
_E = 320000
_NC = 2
_NS = 16
_NW = _NC * _NS
_K = 80
_NBD = _E // (_NW * _K)
_NB = _E // (_NS * _K)
_NP = 10240
_RPT = _NP // _NS
_ZR = 128
_DW = 8
_RB = 1000


def _make_deg_kernel():
    mesh = plsc.VectorSubcoreMesh(core_axis_name="c", subcore_axis_name="s")

    @functools.partial(
        pl.kernel,
        out_type=jax.ShapeDtypeStruct((_NC, _NP, _DW), jnp.float32),
        mesh=mesh,
        compiler_params=pltpu.CompilerParams(use_tc_tiling_on_sc=False),
        scratch_types=[
            pltpu.VMEM((_NBD, _K), jnp.int32),
            pltpu.VMEM((_K + _ZR, _DW), jnp.float32),
            pltpu.VMEM_SHARED((_NP, _DW), jnp.float32),
        ],
    )
    def deg_kernel(const_hbm, dst_hbm, out_hbm, dst_v, const_v, acc):
        cid = lax.axis_index("c")
        sid = lax.axis_index("s")
        wid = cid * _NS + sid
        pltpu.sync_copy(dst_hbm.at[wid], dst_v)
        pltpu.sync_copy(const_hbm, const_v)
        ones_v = const_v.at[pl.ds(0, _K)]
        zero_v = const_v.at[pl.ds(_K, _ZR)]

        base = sid * _RPT
        for j in range(_RPT // _ZR):
            pltpu.sync_copy(zero_v, acc.at[pl.ds(base + j * _ZR, _ZR)])
        plsc.subcore_barrier()

        def body(b, carry):
            pltpu.sync_copy(ones_v, acc.at[dst_v.at[b]], add=True)
            return carry

        lax.fori_loop(0, _NBD, body, 0)
        plsc.subcore_barrier()
        pltpu.sync_copy(acc.at[pl.ds(base, _RPT)],
                        out_hbm.at[cid, pl.ds(base, _RPT)])

    return deg_kernel


def _newton_rsqrt(x):
    i = plsc.bitcast(x, jnp.int32)
    i = 0x5F3759DF - lax.shift_right_logical(i, 1)
    y = plsc.bitcast(i, jnp.float32)
    for _ in range(3):
        y = y * (1.5 - 0.5 * x * y * y)
    return y


def _make_spmm_kernel(d, nbuf, pref):
    drain = nbuf - pref
    mesh = plsc.VectorSubcoreMesh(core_axis_name="c", subcore_axis_name="s")

    @functools.partial(
        pl.kernel,
        out_type=jax.ShapeDtypeStruct((_NC, _NP, d), jnp.float32),
        mesh=mesh,
        compiler_params=pltpu.CompilerParams(use_tc_tiling_on_sc=False),
        scratch_types=[
            pltpu.VMEM((_NB, _K), jnp.int32),
            pltpu.VMEM((_NB, _K), jnp.int32),
            [pltpu.VMEM((_K, d), jnp.float32)] * nbuf,
            pltpu.VMEM((_ZR, d), jnp.float32),
            pltpu.VMEM_SHARED((_NP, d), jnp.float32),
            [pltpu.SemaphoreType.DMA] * nbuf,
            [pltpu.SemaphoreType.DMA] * nbuf,
        ],
    )
    def spmm(x_hbm, src_hbm, dst_hbm, out_hbm,
             src_v, dst_v, bufs, zero_v, acc,
             gsems, ssems):
        cid = lax.axis_index("c")
        sid = lax.axis_index("s")
        tab = x_hbm.at[cid]
        pltpu.sync_copy(src_hbm.at[sid], src_v)
        pltpu.sync_copy(dst_hbm.at[sid], dst_v)

        zer = jnp.zeros((16,), jnp.float32)

        def zfill(r, carry):
            for c in range(d // 16):
                zero_v[r, pl.ds(c * 16, 16)] = zer
            return carry

        lax.fori_loop(0, _ZR, zfill, 0)

        base = sid * _RPT
        for j in range(_RPT // _ZR):
            pltpu.sync_copy(zero_v, acc.at[pl.ds(base + j * _ZR, _ZR)])
        plsc.subcore_barrier()

        def start_g(b, j):
            pltpu.async_copy(tab.at[src_v.at[b]], bufs[j], gsems[j])

        def wait_g(b, j):
            pltpu.make_async_copy(tab.at[src_v.at[b]], bufs[j],
                                  gsems[j]).wait()

        def start_s(b, j):
            pltpu.async_copy(bufs[j], acc.at[dst_v.at[b]], ssems[j],
                             add=True)

        def wait_s(j):
            pltpu.make_async_copy(bufs[j], acc.at[dst_v.at[0]],
                                  ssems[j]).wait()

        for b in range(pref):
            start_g(b, b)
        for b in range(drain):
            wait_g(b, b)
            start_s(b, b)
            start_g(b + pref, b + pref)

        def body(i, carry):
            b0 = nbuf * i + drain
            for k in range(nbuf):
                b = b0 + k
                j = (drain + k) % nbuf
                j2 = (drain + k + pref) % nbuf
                wait_g(b, j)
                start_s(b, j)
                wait_s(j2)
                start_g(b + pref, j2)
            return carry

        lax.fori_loop(0, (_NB - nbuf) // nbuf, body, 0)

        for b in range(_NB - pref, _NB):
            j = b % nbuf
            j2 = (b + pref) % nbuf
            wait_g(b, j)
            start_s(b, j)
            wait_s(j2)
        for b in range(_NB - drain, _NB):
            wait_s(b % nbuf)

        plsc.subcore_barrier()
        pltpu.sync_copy(acc.at[pl.ds(base, _RPT)],
                        out_hbm.at[cid, pl.ds(base, _RPT)])

    return spmm


def _make_l1_fused_kernel(nbuf, pref):
    d = 64
    drain = nbuf - pref
    mesh = plsc.VectorSubcoreMesh(core_axis_name="c", subcore_axis_name="s")

    @functools.partial(
        pl.kernel,
        out_type=(
            jax.ShapeDtypeStruct((_NC, _NP, d), jnp.float32),
            jax.ShapeDtypeStruct((_NC, _N, d), jnp.float32),
            jax.ShapeDtypeStruct((_N,), jnp.float32),
        ),
        mesh=mesh,
        compiler_params=pltpu.CompilerParams(
            use_tc_tiling_on_sc=False, needs_layout_passes=False),
        scratch_types=[
            pltpu.VMEM((_NB, _K), jnp.int32),
            pltpu.VMEM((_NB, _K), jnp.int32),
            [pltpu.VMEM((_K, d), jnp.float32)] * nbuf,
            pltpu.VMEM((_ZR, d), jnp.float32),
            pltpu.VMEM((80, _DW), jnp.float32),
            pltpu.VMEM((80, _DW), jnp.float32),
            pltpu.VMEM((80, d), jnp.float32),
            pltpu.VMEM((80,), jnp.float32),
            pltpu.VMEM_SHARED((_NP, d), jnp.float32),
            [pltpu.SemaphoreType.DMA] * nbuf,
            [pltpu.SemaphoreType.DMA] * nbuf,
        ],
    )
    def spmm(degp_hbm, stc_hbm, src_hbm, dst_hbm,
             p_hbm, xs2_hbm, dinv_hbm,
             src_v, dst_v, bufs, zero_v, d0buf, d1buf, stcbuf, dinvbuf,
             acc, gsems, ssems):
        cid = lax.axis_index("c")
        sid = lax.axis_index("s")
        tab = xs2_hbm.at[cid]
        pltpu.sync_copy(src_hbm.at[sid], src_v)
        pltpu.sync_copy(dst_hbm.at[sid], dst_v)

        zer = jnp.zeros((16,), jnp.float32)

        def zfill(r, carry):
            for c in range(d // 16):
                zero_v[r, pl.ds(c * 16, 16)] = zer
            return carry

        lax.fori_loop(0, _ZR, zfill, 0)

        base = sid * _RPT
        for j in range(_RPT // _ZR):
            pltpu.sync_copy(zero_v, acc.at[pl.ds(base + j * _ZR, _ZR)])

        zidx = jnp.zeros((16,), jnp.int32)
        nchunks = jnp.where(sid == _NS - 1, 5, 8)

        def chunk(cc, carry):
            row0 = pl.multiple_of(base + cc * 80, 16)
            pltpu.sync_copy(degp_hbm.at[0, pl.ds(row0, 80)], d0buf)
            pltpu.sync_copy(degp_hbm.at[1, pl.ds(row0, 80)], d1buf)

            @pl.when(cid == 0)
            def _():
                pltpu.sync_copy(
                    stc_hbm.at[pl.ds(row0, 80), pl.ds(0, d)], stcbuf)

            @pl.when(cid == 1)
            def _():
                pltpu.sync_copy(
                    stc_hbm.at[pl.ds(row0, 80), pl.ds(d, d)], stcbuf)

            for v in range(5):
                ridx = lax.iota(jnp.int32, 16) + (16 * v)
                d0 = plsc.load_gather(d0buf, [ridx, zidx])
                d1 = plsc.load_gather(d1buf, [ridx, zidx])
                y = _newton_rsqrt(d0 + d1 + 1.0)
                dinvbuf[pl.ds(16 * v, 16)] = y
                for r in range(16):
                    rr = 16 * v + r
                    bro = plsc.load_gather(
                        dinvbuf, [jnp.full((16,), rr, jnp.int32)])
                    for c in range(d // 16):
                        sl = pl.ds(c * 16, 16)
                        stcbuf[rr, sl] = stcbuf[rr, sl] * bro

            pltpu.sync_copy(stcbuf, tab.at[pl.ds(row0, 80)])

            @pl.when(cid == 0)
            def _():
                pltpu.sync_copy(dinvbuf, dinv_hbm.at[pl.ds(row0, 80)])

            return carry

        lax.fori_loop(0, nchunks, chunk, 0)
        plsc.subcore_barrier()

        def start_g(b, j):
            pltpu.async_copy(tab.at[src_v.at[b]], bufs[j], gsems[j])

        def wait_g(b, j):
            pltpu.make_async_copy(tab.at[src_v.at[b]], bufs[j],
                                  gsems[j]).wait()

        def start_s(b, j):
            pltpu.async_copy(bufs[j], acc.at[dst_v.at[b]], ssems[j],
                             add=True)

        def wait_s(j):
            pltpu.make_async_copy(bufs[j], acc.at[dst_v.at[0]],
                                  ssems[j]).wait()

        for b in range(pref):
            start_g(b, b)
        for b in range(drain):
            wait_g(b, b)
            start_s(b, b)
            start_g(b + pref, b + pref)

        def body(i, carry):
            b0 = nbuf * i + drain
            for kk in range(nbuf):
                b = b0 + kk
                j = (drain + kk) % nbuf
                j2 = (drain + kk + pref) % nbuf
                wait_g(b, j)
                start_s(b, j)
                wait_s(j2)
                start_g(b + pref, j2)
            return carry

        lax.fori_loop(0, (_NB - nbuf) // nbuf, body, 0)

        for b in range(_NB - pref, _NB):
            j = b % nbuf
            j2 = (b + pref) % nbuf
            wait_g(b, j)
            start_s(b, j)
            wait_s(j2)
        for b in range(_NB - drain, _NB):
            wait_s(b % nbuf)

        plsc.subcore_barrier()
        pltpu.sync_copy(acc.at[pl.ds(base, _RPT)],
                        p_hbm.at[cid, pl.ds(base, _RPT)])

    return spmm


_deg_call = _make_deg_kernel()
_l1_fused_call = _make_l1_fused_kernel(5, 3)
_spmm_l2_call = _make_spmm_kernel(32, 10, 6)


def _prep_call(degp, stc):
    def body(degp_ref, stc_ref, dinv_ref, xs2_ref):
        deg = degp_ref[0, :, :1] + degp_ref[1, :, :1] + 1.0
        dinv = lax.rsqrt(deg)
        dinv_ref[...] = dinv
        xs = stc_ref[...] * dinv
        xs2_ref[0] = xs[:, :64]
        xs2_ref[1] = xs[:, 64:]

    return pl.pallas_call(
        body,
        grid=(_N // _RB,),
        in_specs=[
            pl.BlockSpec((_NC, _RB, _DW), lambda i: (0, i, 0)),
            pl.BlockSpec((_RB, 128), lambda i: (i, 0)),
        ],
        out_specs=[
            pl.BlockSpec((_RB, 1), lambda i: (i, 0)),
            pl.BlockSpec((_NC, _RB, 64), lambda i: (0, i, 0)),
        ],
        out_shape=[
            jax.ShapeDtypeStruct((_N, 1), jnp.float32),
            jax.ShapeDtypeStruct((_NC, _N, 64), jnp.float32),
        ],
    )(degp, stc)


def _mid_call(p, xs2, dinv, w1, b1, w2):
    def body(p_ref, xs_ref, dinv_ref, w1_ref, b1_ref, w2_ref, ms2_ref):
        pfull = jnp.concatenate([p_ref[0], p_ref[1]], axis=-1)
        xfull = jnp.concatenate([xs_ref[0], xs_ref[1]], axis=-1)
        t = (pfull + xfull) * dinv_ref[...]
        h1 = jnp.dot(t, w1_ref[...],
                     preferred_element_type=jnp.float32) + b1_ref[...]
        m = jnp.dot(h1, w2_ref[...], preferred_element_type=jnp.float32)
        ms = m * dinv_ref[...]
        ms2_ref[0] = ms[:, :32]
        ms2_ref[1] = ms[:, 32:]

    return pl.pallas_call(
        body,
        grid=(_N // _RB,),
        in_specs=[
            pl.BlockSpec((_NC, _RB, 64), lambda i: (0, i, 0)),
            pl.BlockSpec((_NC, _RB, 64), lambda i: (0, i, 0)),
            pl.BlockSpec((_RB, 1), lambda i: (i, 0)),
            pl.BlockSpec((128, 256), lambda i: (0, 0)),
            pl.BlockSpec((1, 256), lambda i: (0, 0)),
            pl.BlockSpec((256, 64), lambda i: (0, 0)),
        ],
        out_specs=pl.BlockSpec((_NC, _RB, 32), lambda i: (0, i, 0)),
        out_shape=jax.ShapeDtypeStruct((_NC, _N, 32), jnp.float32),
    )(p, xs2, dinv, w1, b1, w2)


def _final_call(q, ms2, dinv, b2, emb_a, wca, wcb, bc):
    def body(q_ref, ms_ref, dinv_ref, b2_ref, emb_ref, wca_ref, wcb_ref,
             bc_ref, out_ref):
        qfull = jnp.concatenate([q_ref[0], q_ref[1]], axis=-1)
        msfull = jnp.concatenate([ms_ref[0], ms_ref[1]], axis=-1)
        y2 = (qfull + msfull) * dinv_ref[...] + b2_ref[...]
        h2 = jnp.maximum(y2, 0.0)
        out_ref[...] = (
            jnp.dot(emb_ref[...], wca_ref[...],
                    preferred_element_type=jnp.float32)
            + jnp.dot(h2, wcb_ref[...], preferred_element_type=jnp.float32)
            + bc_ref[...])

    return pl.pallas_call(
        body,
        grid=(_N // _RB,),
        in_specs=[
            pl.BlockSpec((_NC, _RB, 32), lambda i: (0, i, 0)),
            pl.BlockSpec((_NC, _RB, 32), lambda i: (0, i, 0)),
            pl.BlockSpec((_RB, 1), lambda i: (i, 0)),
            pl.BlockSpec((1, 64), lambda i: (0, 0)),
            pl.BlockSpec((_RB, 64), lambda i: (i, 0)),
            pl.BlockSpec((64, 64), lambda i: (0, 0)),
            pl.BlockSpec((64, 64), lambda i: (0, 0)),
            pl.BlockSpec((1, 64), lambda i: (0, 0)),
        ],
        out_specs=pl.BlockSpec((_RB, 64), lambda i: (i, 0)),
        out_shape=jax.ShapeDtypeStruct((_N, 64), jnp.float32),
    )(q, ms2, dinv, b2, emb_a, wca, wcb, bc)


def kernel(x, stc_enc, emb_a, W1, b1, W2, b2, Wc, bc, edge_index):
    dst32 = edge_index[1].reshape(_NW, _NBD, _K)
    src16 = edge_index[0].reshape(_NS, _NB, _K)
    dst16 = edge_index[1].reshape(_NS, _NB, _K)
    const = jnp.zeros((_K + _ZR, _DW), jnp.float32).at[:_K, 0].set(1.0)
    degp = _deg_call(const, dst32)
    p, xs2, dinv1 = _l1_fused_call(degp, stc_enc, src16, dst16)
    dinv = dinv1.reshape(_N, 1)
    ms2 = _mid_call(p, xs2, dinv, W1, b1.reshape(1, -1), W2)
    q = _spmm_l2_call(ms2, src16, dst16)
    return _final_call(q, ms2, dinv, b2.reshape(1, -1), emb_a,
                       Wc[:64], Wc[64:], bc.reshape(1, -1))


# speedup vs baseline: 1.0460x; 1.0460x over previous
import functools

import jax
import jax.numpy as jnp
from jax import lax
from jax.experimental import pallas as pl
from jax.experimental.pallas import tpu as pltpu
from jax.experimental.pallas import tpu_sc as plsc

_N = 10000
_E = 320000
_NC = 2
_NS = 16
_NW = _NC * _NS
_K = 80
_NBD = _E // (_NW * _K)
_NB = _E // (_NS * _K)
_NP = 10240
_RPT = _NP // _NS
_ZR = 128
_DW = 8
_RB = 1000


def _make_deg_kernel():
    mesh = plsc.VectorSubcoreMesh(core_axis_name="c", subcore_axis_name="s")

    @functools.partial(
        pl.kernel,
        out_type=jax.ShapeDtypeStruct((_NC, _NP, _DW), jnp.float32),
        mesh=mesh,
        compiler_params=pltpu.CompilerParams(use_tc_tiling_on_sc=False),
        scratch_types=[
            pltpu.VMEM((_NBD, _K), jnp.int32),
            pltpu.VMEM((_K + _ZR, _DW), jnp.float32),
            pltpu.VMEM_SHARED((_NP, _DW), jnp.float32),
            pltpu.SemaphoreType.DMA,
        ],
    )
    def deg_kernel(const_hbm, dst_hbm, out_hbm, dst_v, const_v, acc, hsem):
        cid = lax.axis_index("c")
        sid = lax.axis_index("s")
        wid = cid * _NS + sid
        pltpu.sync_copy(dst_hbm.at[wid], dst_v)
        pltpu.sync_copy(const_hbm, const_v)
        ones_v = const_v.at[pl.ds(0, _K)]
        zero_v = const_v.at[pl.ds(_K, _ZR)]

        base = sid * _RPT
        for j in range(_RPT // _ZR):
            pltpu.sync_copy(zero_v, acc.at[pl.ds(base + j * _ZR, _ZR)])
        plsc.subcore_barrier()

        def fire(b):
            pltpu.async_copy(ones_v, acc.at[dst_v.at[b]], hsem, add=True)

        def drainone():
            pltpu.make_async_copy(ones_v, acc.at[dst_v.at[0]],
                                  hsem).wait()

        for b in range(4):
            fire(b)

        def body(b, carry):
            drainone()
            fire(b + 4)
            return carry

        lax.fori_loop(0, _NBD - 4, body, 0)
        for _ in range(4):
            drainone()
        plsc.subcore_barrier()
        pltpu.sync_copy(acc.at[pl.ds(base, _RPT)],
                        out_hbm.at[cid, pl.ds(base, _RPT)])

    return deg_kernel


def _make_spmm_kernel(d, nbuf, pref):
    drain = nbuf - pref
    mesh = plsc.VectorSubcoreMesh(core_axis_name="c", subcore_axis_name="s")

    @functools.partial(
        pl.kernel,
        out_type=jax.ShapeDtypeStruct((_NC, _NP, d), jnp.float32),
        mesh=mesh,
        compiler_params=pltpu.CompilerParams(use_tc_tiling_on_sc=False),
        scratch_types=[
            pltpu.VMEM((_NB, _K), jnp.int32),
            pltpu.VMEM((_NB, _K), jnp.int32),
            [pltpu.VMEM((_K, d), jnp.float32)] * nbuf,
            pltpu.VMEM((_ZR, d), jnp.float32),
            pltpu.VMEM_SHARED((_NP, d), jnp.float32),
            [pltpu.SemaphoreType.DMA] * nbuf,
            [pltpu.SemaphoreType.DMA] * nbuf,
        ],
    )
    def spmm(x_hbm, src_hbm, dst_hbm, out_hbm,
             src_v, dst_v, bufs, zero_v, acc,
             gsems, ssems):
        cid = lax.axis_index("c")
        sid = lax.axis_index("s")
        tab = x_hbm.at[cid]
        pltpu.sync_copy(src_hbm.at[sid], src_v)
        pltpu.sync_copy(dst_hbm.at[sid], dst_v)

        zer = jnp.zeros((16,), jnp.float32)

        def zfill(r, carry):
            for c in range(d // 16):
                zero_v[r, pl.ds(c * 16, 16)] = zer
            return carry

        lax.fori_loop(0, _ZR, zfill, 0)

        base = sid * _RPT
        for j in range(_RPT // _ZR):
            pltpu.sync_copy(zero_v, acc.at[pl.ds(base + j * _ZR, _ZR)])
        plsc.subcore_barrier()

        def start_g(b, j):
            pltpu.async_copy(tab.at[src_v.at[b]], bufs[j], gsems[j])

        def wait_g(b, j):
            pltpu.make_async_copy(tab.at[src_v.at[b]], bufs[j],
                                  gsems[j]).wait()

        def start_s(b, j):
            pltpu.async_copy(bufs[j], acc.at[dst_v.at[b]], ssems[j],
                             add=True)

        def wait_s(j):
            pltpu.make_async_copy(bufs[j], acc.at[dst_v.at[0]],
                                  ssems[j]).wait()

        for b in range(pref):
            start_g(b, b)
        for b in range(drain):
            wait_g(b, b)
            start_s(b, b)
            start_g(b + pref, b + pref)

        def body(i, carry):
            b0 = nbuf * i + drain
            for k in range(nbuf):
                b = b0 + k
                j = (drain + k) % nbuf
                j2 = (drain + k + pref) % nbuf
                wait_g(b, j)
                start_s(b, j)
                wait_s(j2)
                start_g(b + pref, j2)
            return carry

        lax.fori_loop(0, (_NB - nbuf) // nbuf, body, 0)

        for b in range(_NB - pref, _NB):
            j = b % nbuf
            j2 = (b + pref) % nbuf
            wait_g(b, j)
            start_s(b, j)
            wait_s(j2)
        for b in range(_NB - drain, _NB):
            wait_s(b % nbuf)

        plsc.subcore_barrier()
        pltpu.sync_copy(acc.at[pl.ds(base, _RPT)],
                        out_hbm.at[cid, pl.ds(base, _RPT)])

    return spmm


_deg_call = _make_deg_kernel()
_spmm_l1_call = _make_spmm_kernel(64, 5, 3)
_spmm_l2_call = _make_spmm_kernel(32, 10, 6)


def _prep_call(degp, stc):
    def body(degp_ref, stc_ref, dinv_ref, xs2_ref):
        deg = degp_ref[0, :, :1] + degp_ref[1, :, :1] + 1.0
        dinv = lax.rsqrt(deg)
        dinv_ref[...] = dinv
        xs = stc_ref[...] * dinv
        xs2_ref[0] = xs[:, :64]
        xs2_ref[1] = xs[:, 64:]

    return pl.pallas_call(
        body,
        grid=(_N // _RB,),
        in_specs=[
            pl.BlockSpec((_NC, _RB, _DW), lambda i: (0, i, 0)),
            pl.BlockSpec((_RB, 128), lambda i: (i, 0)),
        ],
        out_specs=[
            pl.BlockSpec((_RB, 1), lambda i: (i, 0)),
            pl.BlockSpec((_NC, _RB, 64), lambda i: (0, i, 0)),
        ],
        out_shape=[
            jax.ShapeDtypeStruct((_N, 1), jnp.float32),
            jax.ShapeDtypeStruct((_NC, _N, 64), jnp.float32),
        ],
    )(degp, stc)


def _mid_call(p, xs2, dinv, w1, b1, w2):
    def body(p_ref, xs_ref, dinv_ref, w1_ref, b1_ref, w2_ref, ms2_ref):
        pfull = jnp.concatenate([p_ref[0], p_ref[1]], axis=-1)
        xfull = jnp.concatenate([xs_ref[0], xs_ref[1]], axis=-1)
        t = (pfull + xfull) * dinv_ref[...]
        h1 = jnp.dot(t, w1_ref[...],
                     preferred_element_type=jnp.float32) + b1_ref[...]
        m = jnp.dot(h1, w2_ref[...], preferred_element_type=jnp.float32)
        ms = m * dinv_ref[...]
        ms2_ref[0] = ms[:, :32]
        ms2_ref[1] = ms[:, 32:]

    return pl.pallas_call(
        body,
        grid=(_N // _RB,),
        in_specs=[
            pl.BlockSpec((_NC, _RB, 64), lambda i: (0, i, 0)),
            pl.BlockSpec((_NC, _RB, 64), lambda i: (0, i, 0)),
            pl.BlockSpec((_RB, 1), lambda i: (i, 0)),
            pl.BlockSpec((128, 256), lambda i: (0, 0)),
            pl.BlockSpec((1, 256), lambda i: (0, 0)),
            pl.BlockSpec((256, 64), lambda i: (0, 0)),
        ],
        out_specs=pl.BlockSpec((_NC, _RB, 32), lambda i: (0, i, 0)),
        out_shape=jax.ShapeDtypeStruct((_NC, _N, 32), jnp.float32),
    )(p, xs2, dinv, w1, b1, w2)


def _final_call(q, ms2, dinv, b2, emb_a, wca, wcb, bc):
    def body(q_ref, ms_ref, dinv_ref, b2_ref, emb_ref, wca_ref, wcb_ref,
             bc_ref, out_ref):
        qfull = jnp.concatenate([q_ref[0], q_ref[1]], axis=-1)
        msfull = jnp.concatenate([ms_ref[0], ms_ref[1]], axis=-1)
        y2 = (qfull + msfull) * dinv_ref[...] + b2_ref[...]
        h2 = jnp.maximum(y2, 0.0)
        out_ref[...] = (
            jnp.dot(emb_ref[...], wca_ref[...],
                    preferred_element_type=jnp.float32)
            + jnp.dot(h2, wcb_ref[...], preferred_element_type=jnp.float32)
            + bc_ref[...])

    return pl.pallas_call(
        body,
        grid=(_N // _RB,),
        in_specs=[
            pl.BlockSpec((_NC, _RB, 32), lambda i: (0, i, 0)),
            pl.BlockSpec((_NC, _RB, 32), lambda i: (0, i, 0)),
            pl.BlockSpec((_RB, 1), lambda i: (i, 0)),
            pl.BlockSpec((1, 64), lambda i: (0, 0)),
            pl.BlockSpec((_RB, 64), lambda i: (i, 0)),
            pl.BlockSpec((64, 64), lambda i: (0, 0)),
            pl.BlockSpec((64, 64), lambda i: (0, 0)),
            pl.BlockSpec((1, 64), lambda i: (0, 0)),
        ],
        out_specs=pl.BlockSpec((_RB, 64), lambda i: (i, 0)),
        out_shape=jax.ShapeDtypeStruct((_N, 64), jnp.float32),
    )(q, ms2, dinv, b2, emb_a, wca, wcb, bc)


def kernel(x, stc_enc, emb_a, W1, b1, W2, b2, Wc, bc, edge_index):
    dst32 = edge_index[1].reshape(_NW, _NBD, _K)
    src16 = edge_index[0].reshape(_NS, _NB, _K)
    dst16 = edge_index[1].reshape(_NS, _NB, _K)
    const = jnp.zeros((_K + _ZR, _DW), jnp.float32).at[:_K, 0].set(1.0)
    degp = _deg_call(const, dst32)
    dinv, xs2 = _prep_call(degp, stc_enc)
    p = _spmm_l1_call(xs2, src16, dst16)
    ms2 = _mid_call(p, xs2, dinv, W1, b1.reshape(1, -1), W2)
    q = _spmm_l2_call(ms2, src16, dst16)
    return _final_call(q, ms2, dinv, b2.reshape(1, -1), emb_a,
                       Wc[:64], Wc[64:], bc.reshape(1, -1))
